# Initial kernel scaffold; baseline (speedup 1.0000x reference)
import jax
import jax.numpy as jnp
from jax.experimental import pallas as pl


def kernel(x, edge_index, reranker_scores, W_l, b_l, W_r, W_s, b_s, alpha):
    raise NotImplementedError("write your pallas kernel here")


# speedup vs baseline: 2.9792x; 2.9792x over previous
import functools

import jax
import jax.numpy as jnp
from jax import lax
from jax.experimental import pallas as pl
from jax.experimental.pallas import tpu as pltpu
from jax.experimental.pallas import tpu_sc as plsc

N_NODES = 10000
N_EDGES = 320000
D = 128

NC = 2
NS = 16
NW = NC * NS
CHUNK = 64
EPT = 10240
NCH = EPT // CHUNK
E_PAD = NW * EPT
TROWS = 4096
RPT = 4000
NTB = 3
TRASH = 4000
DROWS = 128
EREP = 64
ZC = TROWS // NS // CHUNK
DZR = DROWS // NS


def _sc_segment_sum(x, eyer, srcm, dstm):
  mesh = plsc.VectorSubcoreMesh(core_axis_name="c", subcore_axis_name="s")

  @functools.partial(
      pl.kernel,
      out_type=(
          jax.ShapeDtypeStruct((NC, NTB, TROWS, D), jnp.float32),
          jax.ShapeDtypeStruct((NC, DROWS, D), jnp.float32),
      ),
      mesh=mesh,
      scratch_types=[
          pltpu.VMEM_SHARED((TROWS, D), jnp.float32),
          pltpu.VMEM_SHARED((TROWS, D), jnp.float32),
          pltpu.VMEM_SHARED((TROWS, D), jnp.float32),
          pltpu.VMEM_SHARED((DROWS, D), jnp.float32),
          pltpu.VMEM((CHUNK,), jnp.int32),
          pltpu.VMEM((CHUNK,), jnp.int32),
          pltpu.VMEM((CHUNK,), jnp.int32),
          pltpu.VMEM((CHUNK,), jnp.int32),
          pltpu.VMEM((CHUNK,), jnp.int32),
          pltpu.VMEM((CHUNK,), jnp.int32),
          pltpu.VMEM((CHUNK, D), jnp.float32),
          pltpu.VMEM((CHUNK, D), jnp.float32),
          pltpu.SemaphoreType.DMA,
      ],
  )
  def k(x_hbm, eyer_hbm, srcm_hbm, dstm_hbm, acc_out, deg_out,
        t0, t1, t2, td, src_c, dst_c, x0, x1, x2, xd, rows, aux, sem0):
    c = lax.axis_index("c")
    s = lax.axis_index("s")
    g = c * NS + s
    tabs = (t0, t1, t2)
    tix = (x0, x1, x2)

    def fill(i, _):
      r = i // (D // 16)
      q = i % (D // 16)
      aux[r, pl.ds(q * 16, 16)] = jnp.zeros((16,), jnp.float32)
      return 0
    lax.fori_loop(0, CHUNK * (D // 16), fill, 0)

    for t in tabs:
      for kk in range(ZC):
        lo = (s * ZC + kk) * CHUNK
        pltpu.sync_copy(aux, t.at[pl.ds(lo, CHUNK)])
    pltpu.sync_copy(aux.at[pl.ds(0, DZR)], td.at[pl.ds(s * DZR, DZR)])

    plsc.subcore_barrier()

    def chunk(j, _):
      pltpu.sync_copy(srcm_hbm.at[g, j], src_c)
      pltpu.sync_copy(dstm_hbm.at[g, j], dst_c)
      for v in range(CHUNK // 16):
        d = dst_c[pl.ds(v * 16, 16)]
        one = jnp.ones((16,), jnp.int32)
        zero = jnp.zeros((16,), jnp.int32)
        tv = (jnp.where(d >= RPT, one, zero)
              + jnp.where(d >= 2 * RPT, one, zero))
        spill = TRASH + (d & 63)
        for t in range(NTB):
          tix[t][pl.ds(v * 16, 16)] = jnp.where(tv == t, d - t * RPT, spill)
        lane = lax.broadcasted_iota(jnp.int32, (16,), 0) + v * 16
        xd[pl.ds(v * 16, 16)] = (d & (D - 1)) + lane * D
      pltpu.async_copy(x_hbm.at[src_c], rows, sem0).wait()
      for t in range(NTB):
        pltpu.sync_copy(rows, tabs[t].at[tix[t]], add=True)
      pltpu.async_copy(eyer_hbm.at[xd], aux, sem0).wait()
      for v in range(CHUNK // 16):
        xd[pl.ds(v * 16, 16)] = (
            dst_c[pl.ds(v * 16, 16)] >> (D.bit_length() - 1))
      pltpu.sync_copy(aux, td.at[xd], add=True)
      return 0

    lax.fori_loop(0, NCH, chunk, 0)

    plsc.subcore_barrier()
    for t in range(NTB):
      for kk in range(ZC):
        lo = (s * ZC + kk) * CHUNK
        pltpu.sync_copy(tabs[t].at[pl.ds(lo, CHUNK)], aux)
        pltpu.sync_copy(aux, acc_out.at[c, t, pl.ds(lo, CHUNK)])
    pltpu.sync_copy(td.at[pl.ds(s * DZR, DZR)], aux.at[pl.ds(0, DZR)])
    pltpu.sync_copy(aux.at[pl.ds(0, DZR)], deg_out.at[c, pl.ds(s * DZR, DZR)])

  return k(x, eyer, srcm, dstm)


BLK = 1000


def _tc_body(alpha_ref, bs_ref, acc_ref, deg_ref, x_ref, rr_ref,
             wlt_ref, bl_ref, wrt_ref, ws_ref, o_ref):
  aggsum = acc_ref[0, 0] + acc_ref[1, 0]
  degc = jnp.maximum(deg_ref[...], 1.0)
  hl = jnp.dot(aggsum, wlt_ref[...], preferred_element_type=jnp.float32)
  h = (hl / degc + bl_ref[...]
       + jnp.dot(x_ref[...], wrt_ref[...], preferred_element_type=jnp.float32))
  h = jnp.maximum(h, 0.0)
  gnn = jnp.dot(h, ws_ref[...], preferred_element_type=jnp.float32) + bs_ref[0, 0]
  a = 1.0 / (1.0 + jnp.exp(-alpha_ref[0, 0]))
  o_ref[...] = a * rr_ref[...] + (1.0 - a) * gnn


def _tc_dense(alpha, b_s, acc, deg2, x, rr2, wl_t, bl2, wr_t, ws_col):
  grid = (N_NODES // BLK,)
  return pl.pallas_call(
      _tc_body,
      grid=grid,
      in_specs=[
          pl.BlockSpec(memory_space=pltpu.SMEM),
          pl.BlockSpec(memory_space=pltpu.SMEM),
          pl.BlockSpec((NC, 1, BLK, D), lambda i: (0, i // 4, i % 4, 0)),
          pl.BlockSpec((BLK, 1), lambda i: (i, 0)),
          pl.BlockSpec((BLK, D), lambda i: (i, 0)),
          pl.BlockSpec((BLK, 1), lambda i: (i, 0)),
          pl.BlockSpec((D, D), lambda i: (0, 0)),
          pl.BlockSpec((1, D), lambda i: (0, 0)),
          pl.BlockSpec((D, D), lambda i: (0, 0)),
          pl.BlockSpec((D, 1), lambda i: (0, 0)),
      ],
      out_specs=pl.BlockSpec((BLK, 1), lambda i: (i, 0)),
      out_shape=jax.ShapeDtypeStruct((N_NODES, 1), jnp.float32),
  )(alpha, b_s, acc, deg2, x, rr2, wl_t, bl2, wr_t, ws_col)


@jax.jit
def kernel(x, edge_index, reranker_scores, W_l, b_l, W_r, W_s, b_s, alpha):
  src = edge_index[0].astype(jnp.int32)
  dst = edge_index[1].astype(jnp.int32)
  pad = E_PAD - N_EDGES
  pad_ids = jnp.arange(pad, dtype=jnp.int32)
  src_p = jnp.concatenate([src, (pad_ids * 997) % N_NODES])
  dst_p = jnp.concatenate([dst, N_NODES + (pad_ids % 240)])
  srcm = src_p.reshape(NW, NCH, CHUNK)
  dstm = dst_p.reshape(NW, NCH, CHUNK)
  eyer = jnp.tile(jnp.eye(D, dtype=jnp.float32), (EREP, 1))

  acc, degp = _sc_segment_sum(x, eyer, srcm, dstm)

  deg2 = (degp[0] + degp[1]).reshape(DROWS * D)[:N_NODES].reshape(N_NODES, 1)

  alpha2 = alpha.reshape(1, 1)
  bs2 = b_s.reshape(1, 1)
  rr2 = reranker_scores.reshape(N_NODES, 1)
  out2 = _tc_dense(alpha2, bs2, acc, deg2, x, rr2,
                   W_l.T, b_l.reshape(1, D), W_r.T, W_s.T)
  return out2.reshape(N_NODES)


# speedup vs baseline: 3.8633x; 1.2968x over previous
import functools

import jax
import jax.numpy as jnp
from jax import lax
from jax.experimental import pallas as pl
from jax.experimental.pallas import tpu as pltpu
from jax.experimental.pallas import tpu_sc as plsc

N_NODES = 10000
N_EDGES = 320000
D = 128

NC = 2
NS = 16
NW = NC * NS
CHUNK = 64
EPT = 10240
NCH = EPT // CHUNK
E_PAD = NW * EPT
TROWS = 4096
RPT = 4000
NTB = 3
TRASH = 4000
DROWS = 128
EREP = 64
ZC = TROWS // NS // CHUNK
DZR = DROWS // NS


def _sc_segment_sum(x, eyer, sdm):
  mesh = plsc.VectorSubcoreMesh(core_axis_name="c", subcore_axis_name="s")

  @functools.partial(
      pl.kernel,
      out_type=(
          jax.ShapeDtypeStruct((NC, NTB, TROWS, D), jnp.float32),
          jax.ShapeDtypeStruct((NC, DROWS, D), jnp.float32),
      ),
      mesh=mesh,
      scratch_types=[
          pltpu.VMEM_SHARED((TROWS, D), jnp.float32),
          pltpu.VMEM_SHARED((TROWS, D), jnp.float32),
          pltpu.VMEM_SHARED((TROWS, D), jnp.float32),
          pltpu.VMEM_SHARED((DROWS, D), jnp.float32),
          pltpu.VMEM((2, CHUNK), jnp.int32),
          pltpu.VMEM((CHUNK,), jnp.int32),
          pltpu.VMEM((CHUNK,), jnp.int32),
          pltpu.VMEM((CHUNK,), jnp.int32),
          pltpu.VMEM((CHUNK,), jnp.int32),
          pltpu.VMEM((CHUNK,), jnp.int32),
          pltpu.VMEM((CHUNK, D), jnp.float32),
          pltpu.VMEM((CHUNK, D), jnp.float32),
          pltpu.SemaphoreType.DMA,
          pltpu.SemaphoreType.DMA,
      ],
  )
  def k(x_hbm, eyer_hbm, sdm_hbm, acc_out, deg_out,
        t0, t1, t2, td, sd_c, x0, x1, x2, xd, dsh, rows, aux, sem0, sem1):
    c = lax.axis_index("c")
    s = lax.axis_index("s")
    g = c * NS + s
    tabs = (t0, t1, t2)
    tix = (x0, x1, x2)

    def fill(i, _):
      r = i // (D // 16)
      q = i % (D // 16)
      aux[r, pl.ds(q * 16, 16)] = jnp.zeros((16,), jnp.float32)
      return 0
    lax.fori_loop(0, CHUNK * (D // 16), fill, 0)

    for t in tabs:
      for kk in range(ZC):
        lo = (s * ZC + kk) * CHUNK
        pltpu.sync_copy(aux, t.at[pl.ds(lo, CHUNK)])
    pltpu.sync_copy(aux.at[pl.ds(0, DZR)], td.at[pl.ds(s * DZR, DZR)])

    plsc.subcore_barrier()

    def chunk(j, _):
      pltpu.sync_copy(sdm_hbm.at[g, j], sd_c)
      pltpu.async_copy(x_hbm.at[sd_c.at[0]], rows, sem0)
      for v in range(CHUNK // 16):
        d = sd_c[1, pl.ds(v * 16, 16)]
        one = jnp.ones((16,), jnp.int32)
        zero = jnp.zeros((16,), jnp.int32)
        tv = (jnp.where(d >= RPT, one, zero)
              + jnp.where(d >= 2 * RPT, one, zero))
        spill = TRASH + (d & 63)
        for t in range(NTB):
          tix[t][pl.ds(v * 16, 16)] = jnp.where(tv == t, d - t * RPT, spill)
        lane = lax.broadcasted_iota(jnp.int32, (16,), 0) + v * 16
        xd[pl.ds(v * 16, 16)] = (d & (D - 1)) + lane * D
        dsh[pl.ds(v * 16, 16)] = d >> (D.bit_length() - 1)
      pltpu.async_copy(eyer_hbm.at[xd], aux, sem1)
      pltpu.make_async_copy(x_hbm.at[sd_c.at[0]], rows, sem0).wait()
      for t in range(NTB):
        pltpu.sync_copy(rows, tabs[t].at[tix[t]], add=True)
      pltpu.make_async_copy(eyer_hbm.at[xd], aux, sem1).wait()
      pltpu.sync_copy(aux, td.at[dsh], add=True)
      return 0

    lax.fori_loop(0, NCH, chunk, 0)

    plsc.subcore_barrier()
    for t in range(NTB):
      for kk in range(ZC):
        lo = (s * ZC + kk) * CHUNK
        pltpu.sync_copy(tabs[t].at[pl.ds(lo, CHUNK)], aux)
        pltpu.sync_copy(aux, acc_out.at[c, t, pl.ds(lo, CHUNK)])
    pltpu.sync_copy(td.at[pl.ds(s * DZR, DZR)], aux.at[pl.ds(0, DZR)])
    pltpu.sync_copy(aux.at[pl.ds(0, DZR)], deg_out.at[c, pl.ds(s * DZR, DZR)])

  return k(x, eyer, sdm)


BLK = 1000


def _tc_body(alpha_ref, bs_ref, acc_ref, deg_ref, x_ref, rr_ref,
             wlt_ref, bl_ref, wrt_ref, ws_ref, o_ref):
  aggsum = acc_ref[0, 0] + acc_ref[1, 0]
  degc = jnp.maximum(deg_ref[...], 1.0)
  hl = jnp.dot(aggsum, wlt_ref[...], preferred_element_type=jnp.float32)
  h = (hl / degc + bl_ref[...]
       + jnp.dot(x_ref[...], wrt_ref[...], preferred_element_type=jnp.float32))
  h = jnp.maximum(h, 0.0)
  gnn = jnp.dot(h, ws_ref[...], preferred_element_type=jnp.float32) + bs_ref[0, 0]
  a = 1.0 / (1.0 + jnp.exp(-alpha_ref[0, 0]))
  o_ref[...] = a * rr_ref[...] + (1.0 - a) * gnn


def _tc_dense(alpha, b_s, acc, deg2, x, rr2, wl_t, bl2, wr_t, ws_col):
  grid = (N_NODES // BLK,)
  return pl.pallas_call(
      _tc_body,
      grid=grid,
      in_specs=[
          pl.BlockSpec(memory_space=pltpu.SMEM),
          pl.BlockSpec(memory_space=pltpu.SMEM),
          pl.BlockSpec((NC, 1, BLK, D), lambda i: (0, i // 4, i % 4, 0)),
          pl.BlockSpec((BLK, 1), lambda i: (i, 0)),
          pl.BlockSpec((BLK, D), lambda i: (i, 0)),
          pl.BlockSpec((BLK, 1), lambda i: (i, 0)),
          pl.BlockSpec((D, D), lambda i: (0, 0)),
          pl.BlockSpec((1, D), lambda i: (0, 0)),
          pl.BlockSpec((D, D), lambda i: (0, 0)),
          pl.BlockSpec((D, 1), lambda i: (0, 0)),
      ],
      out_specs=pl.BlockSpec((BLK, 1), lambda i: (i, 0)),
      out_shape=jax.ShapeDtypeStruct((N_NODES, 1), jnp.float32),
  )(alpha, b_s, acc, deg2, x, rr2, wl_t, bl2, wr_t, ws_col)


@jax.jit
def kernel(x, edge_index, reranker_scores, W_l, b_l, W_r, W_s, b_s, alpha):
  src = edge_index[0].astype(jnp.int32)
  dst = edge_index[1].astype(jnp.int32)
  pad = E_PAD - N_EDGES
  pad_ids = jnp.arange(pad, dtype=jnp.int32)
  src_p = jnp.concatenate([src, (pad_ids * 997) % N_NODES])
  dst_p = jnp.concatenate([dst, N_NODES + (pad_ids % 240)])
  sdm = jnp.stack([src_p.reshape(NW, NCH, CHUNK),
                   dst_p.reshape(NW, NCH, CHUNK)], axis=2)
  eyer = jnp.tile(jnp.eye(D, dtype=jnp.float32), (EREP, 1))

  acc, degp = _sc_segment_sum(x, eyer, sdm)

  deg2 = (degp[0] + degp[1]).reshape(DROWS * D)[:N_NODES].reshape(N_NODES, 1)

  alpha2 = alpha.reshape(1, 1)
  bs2 = b_s.reshape(1, 1)
  rr2 = reranker_scores.reshape(N_NODES, 1)
  out2 = _tc_dense(alpha2, bs2, acc, deg2, x, rr2,
                   W_l.T, b_l.reshape(1, D), W_r.T, W_s.T)
  return out2.reshape(N_NODES)


# speedup vs baseline: 4.0804x; 1.0562x over previous
import functools

import jax
import jax.numpy as jnp
from jax import lax
from jax.experimental import pallas as pl
from jax.experimental.pallas import tpu as pltpu
from jax.experimental.pallas import tpu_sc as plsc

N_NODES = 10000
N_EDGES = 320000
D = 128

NC = 2
NS = 16
NW = NC * NS
CHUNK = 64
EPT = 10240
NCH = EPT // CHUNK
E_PAD = NW * EPT
TROWS = 4096
RPT = 4000
NTB = 3
TRASH = 4000
DROWS = 128
EREP = 64
ZC = TROWS // NS // CHUNK
DZR = DROWS // NS


def _sc_segment_sum(x, eyer, sdm):
  mesh = plsc.VectorSubcoreMesh(core_axis_name="c", subcore_axis_name="s")

  @functools.partial(
      pl.kernel,
      out_type=(
          jax.ShapeDtypeStruct((NC, NTB, TROWS, D), jnp.float32),
          jax.ShapeDtypeStruct((NC, DROWS, D), jnp.float32),
      ),
      mesh=mesh,
      scratch_types=[
          pltpu.VMEM_SHARED((TROWS, D), jnp.float32),
          pltpu.VMEM_SHARED((TROWS, D), jnp.float32),
          pltpu.VMEM_SHARED((TROWS, D), jnp.float32),
          pltpu.VMEM_SHARED((DROWS, D), jnp.float32),
          pltpu.VMEM((2, CHUNK), jnp.int32),
          pltpu.VMEM((CHUNK,), jnp.int32),
          pltpu.VMEM((CHUNK,), jnp.int32),
          pltpu.VMEM((CHUNK,), jnp.int32),
          pltpu.VMEM((CHUNK,), jnp.int32),
          pltpu.VMEM((CHUNK,), jnp.int32),
          pltpu.VMEM((CHUNK, D), jnp.float32),
          pltpu.VMEM((CHUNK, D), jnp.float32),
          pltpu.SemaphoreType.DMA,
          pltpu.SemaphoreType.DMA,
          pltpu.SemaphoreType.DMA,
      ],
  )
  def k(x_hbm, eyer_hbm, sdm_hbm, acc_out, deg_out,
        t0, t1, t2, td, sd_c, x0, x1, x2, xd, dsh, rows, aux,
        sem0, sem1, sems):
    c = lax.axis_index("c")
    s = lax.axis_index("s")
    g = c * NS + s
    tabs = (t0, t1, t2)
    tix = (x0, x1, x2)

    def fill(i, _):
      r = i // (D // 16)
      q = i % (D // 16)
      aux[r, pl.ds(q * 16, 16)] = jnp.zeros((16,), jnp.float32)
      return 0
    lax.fori_loop(0, CHUNK * (D // 16), fill, 0)

    for t in tabs:
      for kk in range(ZC):
        lo = (s * ZC + kk) * CHUNK
        pltpu.sync_copy(aux, t.at[pl.ds(lo, CHUNK)])
    pltpu.sync_copy(aux.at[pl.ds(0, DZR)], td.at[pl.ds(s * DZR, DZR)])

    plsc.subcore_barrier()

    def chunk(j, _):
      pltpu.sync_copy(sdm_hbm.at[g, j], sd_c)
      pltpu.async_copy(x_hbm.at[sd_c.at[0]], rows, sem0)
      for v in range(CHUNK // 16):
        d = sd_c[1, pl.ds(v * 16, 16)]
        one = jnp.ones((16,), jnp.int32)
        zero = jnp.zeros((16,), jnp.int32)
        tv = (jnp.where(d >= RPT, one, zero)
              + jnp.where(d >= 2 * RPT, one, zero))
        spill = TRASH + (d & 63)
        for t in range(NTB):
          tix[t][pl.ds(v * 16, 16)] = jnp.where(tv == t, d - t * RPT, spill)
        lane = lax.broadcasted_iota(jnp.int32, (16,), 0) + v * 16
        xd[pl.ds(v * 16, 16)] = (d & (D - 1)) + lane * D
        dsh[pl.ds(v * 16, 16)] = d >> (D.bit_length() - 1)
      pltpu.async_copy(eyer_hbm.at[xd], aux, sem1)
      pltpu.make_async_copy(x_hbm.at[sd_c.at[0]], rows, sem0).wait()
      for t in range(NTB):
        pltpu.async_copy(rows, tabs[t].at[tix[t]], sems, add=True)
      pltpu.make_async_copy(eyer_hbm.at[xd], aux, sem1).wait()
      pltpu.async_copy(aux, td.at[dsh], sems, add=True)
      for t in range(NTB):
        pltpu.make_async_copy(rows, tabs[t].at[tix[t]], sems).wait()
      pltpu.make_async_copy(aux, td.at[dsh], sems).wait()
      return 0

    lax.fori_loop(0, NCH, chunk, 0)

    plsc.subcore_barrier()
    for t in range(NTB):
      for kk in range(ZC):
        lo = (s * ZC + kk) * CHUNK
        pltpu.sync_copy(tabs[t].at[pl.ds(lo, CHUNK)], aux)
        pltpu.sync_copy(aux, acc_out.at[c, t, pl.ds(lo, CHUNK)])
    pltpu.sync_copy(td.at[pl.ds(s * DZR, DZR)], aux.at[pl.ds(0, DZR)])
    pltpu.sync_copy(aux.at[pl.ds(0, DZR)], deg_out.at[c, pl.ds(s * DZR, DZR)])

  return k(x, eyer, sdm)


BLK = 1000


def _tc_body(alpha_ref, bs_ref, acc_ref, deg_ref, x_ref, rr_ref,
             wlt_ref, bl_ref, wrt_ref, ws_ref, o_ref):
  aggsum = acc_ref[0, 0] + acc_ref[1, 0]
  degc = jnp.maximum(deg_ref[...], 1.0)
  hl = jnp.dot(aggsum, wlt_ref[...], preferred_element_type=jnp.float32)
  h = (hl / degc + bl_ref[...]
       + jnp.dot(x_ref[...], wrt_ref[...], preferred_element_type=jnp.float32))
  h = jnp.maximum(h, 0.0)
  gnn = jnp.dot(h, ws_ref[...], preferred_element_type=jnp.float32) + bs_ref[0, 0]
  a = 1.0 / (1.0 + jnp.exp(-alpha_ref[0, 0]))
  o_ref[...] = a * rr_ref[...] + (1.0 - a) * gnn


def _tc_dense(alpha, b_s, acc, deg2, x, rr2, wl_t, bl2, wr_t, ws_col):
  grid = (N_NODES // BLK,)
  return pl.pallas_call(
      _tc_body,
      grid=grid,
      in_specs=[
          pl.BlockSpec(memory_space=pltpu.SMEM),
          pl.BlockSpec(memory_space=pltpu.SMEM),
          pl.BlockSpec((NC, 1, BLK, D), lambda i: (0, i // 4, i % 4, 0)),
          pl.BlockSpec((BLK, 1), lambda i: (i, 0)),
          pl.BlockSpec((BLK, D), lambda i: (i, 0)),
          pl.BlockSpec((BLK, 1), lambda i: (i, 0)),
          pl.BlockSpec((D, D), lambda i: (0, 0)),
          pl.BlockSpec((1, D), lambda i: (0, 0)),
          pl.BlockSpec((D, D), lambda i: (0, 0)),
          pl.BlockSpec((D, 1), lambda i: (0, 0)),
      ],
      out_specs=pl.BlockSpec((BLK, 1), lambda i: (i, 0)),
      out_shape=jax.ShapeDtypeStruct((N_NODES, 1), jnp.float32),
  )(alpha, b_s, acc, deg2, x, rr2, wl_t, bl2, wr_t, ws_col)


@jax.jit
def kernel(x, edge_index, reranker_scores, W_l, b_l, W_r, W_s, b_s, alpha):
  src = edge_index[0].astype(jnp.int32)
  dst = edge_index[1].astype(jnp.int32)
  pad = E_PAD - N_EDGES
  pad_ids = jnp.arange(pad, dtype=jnp.int32)
  src_p = jnp.concatenate([src, (pad_ids * 997) % N_NODES])
  dst_p = jnp.concatenate([dst, N_NODES + (pad_ids % 240)])
  sdm = jnp.stack([src_p.reshape(NW, NCH, CHUNK),
                   dst_p.reshape(NW, NCH, CHUNK)], axis=2)
  eyer = jnp.tile(jnp.eye(D, dtype=jnp.float32), (EREP, 1))

  acc, degp = _sc_segment_sum(x, eyer, sdm)

  deg2 = (degp[0] + degp[1]).reshape(DROWS * D)[:N_NODES].reshape(N_NODES, 1)

  alpha2 = alpha.reshape(1, 1)
  bs2 = b_s.reshape(1, 1)
  rr2 = reranker_scores.reshape(N_NODES, 1)
  out2 = _tc_dense(alpha2, bs2, acc, deg2, x, rr2,
                   W_l.T, b_l.reshape(1, D), W_r.T, W_s.T)
  return out2.reshape(N_NODES)


# speedup vs baseline: 5.9959x; 1.4694x over previous
import functools

import jax
import jax.numpy as jnp
from jax import lax
from jax.experimental import pallas as pl
from jax.experimental.pallas import tpu as pltpu
from jax.experimental.pallas import tpu_sc as plsc

N_NODES = 10000
N_EDGES = 320000
D = 128

NC = 2
NS = 16
NW = NC * NS
CHUNK = 64
EPT = 10240
NCH = EPT // CHUNK
E_PAD = NW * EPT
TROWS = 4096
RPT = 4000
NTB = 3
TRASH = 4000
DROWS = 128
EREP = 64
ZC = TROWS // NS // CHUNK
DZR = DROWS // NS


def _sc_segment_sum(x, eyer, sdm):
  mesh = plsc.VectorSubcoreMesh(core_axis_name="c", subcore_axis_name="s")

  @functools.partial(
      pl.kernel,
      out_type=(
          jax.ShapeDtypeStruct((NC, NTB, TROWS, D), jnp.float32),
          jax.ShapeDtypeStruct((NC, DROWS, D), jnp.float32),
      ),
      mesh=mesh,
      scratch_types=[
          pltpu.VMEM_SHARED((TROWS, D), jnp.float32),
          pltpu.VMEM_SHARED((TROWS, D), jnp.float32),
          pltpu.VMEM_SHARED((TROWS, D), jnp.float32),
          pltpu.VMEM_SHARED((DROWS, D), jnp.float32),
          pltpu.VMEM((2, CHUNK), jnp.int32),
          pltpu.VMEM((2, CHUNK), jnp.int32),
          pltpu.VMEM((CHUNK,), jnp.int32),
          pltpu.VMEM((CHUNK,), jnp.int32),
          pltpu.VMEM((CHUNK,), jnp.int32),
          pltpu.VMEM((CHUNK,), jnp.int32),
          pltpu.VMEM((CHUNK,), jnp.int32),
          pltpu.VMEM((CHUNK, D), jnp.float32),
          pltpu.VMEM((CHUNK, D), jnp.float32),
          pltpu.VMEM((CHUNK, D), jnp.float32),
          pltpu.SemaphoreType.DMA,
          pltpu.SemaphoreType.DMA,
          pltpu.SemaphoreType.DMA,
          pltpu.SemaphoreType.DMA,
          pltpu.SemaphoreType.DMA,
          pltpu.SemaphoreType.DMA,
      ],
  )
  def k(x_hbm, eyer_hbm, sdm_hbm, acc_out, deg_out,
        t0, t1, t2, td, sd0, sd1, x0, x1, x2, xd, dsh, rows0, rows1, aux,
        semg0, semg1, semi0, semi1, sem1, sems):
    c = lax.axis_index("c")
    s = lax.axis_index("s")
    g = c * NS + s
    tabs = (t0, t1, t2)
    tix = (x0, x1, x2)

    def fill(i, _):
      r = i // (D // 16)
      q = i % (D // 16)
      aux[r, pl.ds(q * 16, 16)] = jnp.zeros((16,), jnp.float32)
      return 0
    lax.fori_loop(0, CHUNK * (D // 16), fill, 0)

    for t in tabs:
      for kk in range(ZC):
        lo = (s * ZC + kk) * CHUNK
        pltpu.sync_copy(aux, t.at[pl.ds(lo, CHUNK)])
    pltpu.sync_copy(aux.at[pl.ds(0, DZR)], td.at[pl.ds(s * DZR, DZR)])

    plsc.subcore_barrier()

    pltpu.sync_copy(sdm_hbm.at[g, 0], sd0)
    pltpu.async_copy(x_hbm.at[sd0.at[0]], rows0, semg0)
    pltpu.async_copy(sdm_hbm.at[g, 1], sd1, semi1)

    def half(j, sd, rows, semg, sd_n, rows_n, semg_n, semi_n, semi_f):
      for v in range(CHUNK // 16):
        d = sd[1, pl.ds(v * 16, 16)]
        one = jnp.ones((16,), jnp.int32)
        zero = jnp.zeros((16,), jnp.int32)
        tv = (jnp.where(d >= RPT, one, zero)
              + jnp.where(d >= 2 * RPT, one, zero))
        spill = TRASH + (d & 63)
        for t in range(NTB):
          tix[t][pl.ds(v * 16, 16)] = jnp.where(tv == t, d - t * RPT, spill)
        lane = lax.broadcasted_iota(jnp.int32, (16,), 0) + v * 16
        xd[pl.ds(v * 16, 16)] = (d & (D - 1)) + lane * D
        dsh[pl.ds(v * 16, 16)] = d >> (D.bit_length() - 1)
      pltpu.async_copy(eyer_hbm.at[xd], aux, sem1)
      pltpu.make_async_copy(x_hbm.at[sd.at[0]], rows, semg).wait()
      for t in range(NTB):
        pltpu.async_copy(rows, tabs[t].at[tix[t]], sems, add=True)
      pltpu.make_async_copy(sdm_hbm.at[g, j + 1], sd_n, semi_n).wait()
      pltpu.async_copy(x_hbm.at[sd_n.at[0]], rows_n, semg_n)
      pltpu.async_copy(sdm_hbm.at[g, j + 2], sd, semi_f)
      pltpu.make_async_copy(eyer_hbm.at[xd], aux, sem1).wait()
      pltpu.async_copy(aux, td.at[dsh], sems, add=True)
      for t in range(NTB):
        pltpu.make_async_copy(rows, tabs[t].at[tix[t]], sems).wait()
      pltpu.make_async_copy(aux, td.at[dsh], sems).wait()

    def chunk(i, _):
      j0 = 2 * i
      half(j0, sd0, rows0, semg0, sd1, rows1, semg1, semi1, semi0)
      half(j0 + 1, sd1, rows1, semg1, sd0, rows0, semg0, semi0, semi1)
      return 0

    lax.fori_loop(0, NCH // 2, chunk, 0)

    pltpu.make_async_copy(x_hbm.at[sd0.at[0]], rows0, semg0).wait()
    pltpu.make_async_copy(sdm_hbm.at[g, 0], sd1, semi1).wait()

    plsc.subcore_barrier()
    for t in range(NTB):
      for kk in range(ZC):
        lo = (s * ZC + kk) * CHUNK
        pltpu.sync_copy(tabs[t].at[pl.ds(lo, CHUNK)], aux)
        pltpu.sync_copy(aux, acc_out.at[c, t, pl.ds(lo, CHUNK)])
    pltpu.sync_copy(td.at[pl.ds(s * DZR, DZR)], aux.at[pl.ds(0, DZR)])
    pltpu.sync_copy(aux.at[pl.ds(0, DZR)], deg_out.at[c, pl.ds(s * DZR, DZR)])

  return k(x, eyer, sdm)


BLK = 1000


def _tc_body(alpha_ref, bs_ref, acc_ref, deg_ref, x_ref, rr_ref,
             wlt_ref, bl_ref, wrt_ref, ws_ref, o_ref):
  aggsum = acc_ref[0, 0] + acc_ref[1, 0]
  degc = jnp.maximum(deg_ref[...], 1.0)
  hl = jnp.dot(aggsum, wlt_ref[...], preferred_element_type=jnp.float32)
  h = (hl / degc + bl_ref[...]
       + jnp.dot(x_ref[...], wrt_ref[...], preferred_element_type=jnp.float32))
  h = jnp.maximum(h, 0.0)
  gnn = jnp.dot(h, ws_ref[...], preferred_element_type=jnp.float32) + bs_ref[0, 0]
  a = 1.0 / (1.0 + jnp.exp(-alpha_ref[0, 0]))
  o_ref[...] = a * rr_ref[...] + (1.0 - a) * gnn


def _tc_dense(alpha, b_s, acc, deg2, x, rr2, wl_t, bl2, wr_t, ws_col):
  grid = (N_NODES // BLK,)
  return pl.pallas_call(
      _tc_body,
      grid=grid,
      in_specs=[
          pl.BlockSpec(memory_space=pltpu.SMEM),
          pl.BlockSpec(memory_space=pltpu.SMEM),
          pl.BlockSpec((NC, 1, BLK, D), lambda i: (0, i // 4, i % 4, 0)),
          pl.BlockSpec((BLK, 1), lambda i: (i, 0)),
          pl.BlockSpec((BLK, D), lambda i: (i, 0)),
          pl.BlockSpec((BLK, 1), lambda i: (i, 0)),
          pl.BlockSpec((D, D), lambda i: (0, 0)),
          pl.BlockSpec((1, D), lambda i: (0, 0)),
          pl.BlockSpec((D, D), lambda i: (0, 0)),
          pl.BlockSpec((D, 1), lambda i: (0, 0)),
      ],
      out_specs=pl.BlockSpec((BLK, 1), lambda i: (i, 0)),
      out_shape=jax.ShapeDtypeStruct((N_NODES, 1), jnp.float32),
  )(alpha, b_s, acc, deg2, x, rr2, wl_t, bl2, wr_t, ws_col)


@jax.jit
def kernel(x, edge_index, reranker_scores, W_l, b_l, W_r, W_s, b_s, alpha):
  src = edge_index[0].astype(jnp.int32)
  dst = edge_index[1].astype(jnp.int32)
  pad = E_PAD - N_EDGES
  pad_ids = jnp.arange(pad, dtype=jnp.int32)
  src_p = jnp.concatenate([src, (pad_ids * 997) % N_NODES])
  dst_p = jnp.concatenate([dst, N_NODES + (pad_ids % 240)])
  sdm = jnp.stack([src_p.reshape(NW, NCH, CHUNK),
                   dst_p.reshape(NW, NCH, CHUNK)], axis=2)
  sdm = jnp.concatenate([sdm, sdm[:, :2]], axis=1)
  eyer = jnp.tile(jnp.eye(D, dtype=jnp.float32), (EREP, 1))

  acc, degp = _sc_segment_sum(x, eyer, sdm)

  deg2 = (degp[0] + degp[1]).reshape(DROWS * D)[:N_NODES].reshape(N_NODES, 1)

  alpha2 = alpha.reshape(1, 1)
  bs2 = b_s.reshape(1, 1)
  rr2 = reranker_scores.reshape(N_NODES, 1)
  out2 = _tc_dense(alpha2, bs2, acc, deg2, x, rr2,
                   W_l.T, b_l.reshape(1, D), W_r.T, W_s.T)
  return out2.reshape(N_NODES)


# speedup vs baseline: 6.0541x; 1.0097x over previous
import functools

import jax
import jax.numpy as jnp
from jax import lax
from jax.experimental import pallas as pl
from jax.experimental.pallas import tpu as pltpu
from jax.experimental.pallas import tpu_sc as plsc

N_NODES = 10000
N_EDGES = 320000
D = 128

NC = 2
NS = 16
NW = NC * NS
CHUNK = 64
EPT = 10240
NCH = EPT // CHUNK
E_PAD = NW * EPT
TROWS = 4096
RPT = 4000
NTB = 3
TRASH = 4000
DROWS = 128
EREP = 64
ZC = TROWS // NS // CHUNK
DZR = DROWS // NS


def _sc_segment_sum(x, eyer, sdm):
  mesh = plsc.VectorSubcoreMesh(core_axis_name="c", subcore_axis_name="s")

  @functools.partial(
      pl.kernel,
      out_type=(
          jax.ShapeDtypeStruct((NC, NTB, TROWS, D), jnp.float32),
          jax.ShapeDtypeStruct((NC, DROWS, D), jnp.float32),
      ),
      mesh=mesh,
      scratch_types=[
          pltpu.VMEM_SHARED((TROWS, D), jnp.float32),
          pltpu.VMEM_SHARED((TROWS, D), jnp.float32),
          pltpu.VMEM_SHARED((TROWS, D), jnp.float32),
          pltpu.VMEM_SHARED((DROWS, D), jnp.float32),
          pltpu.VMEM((2, CHUNK), jnp.int32),
          pltpu.VMEM((2, CHUNK), jnp.int32),
          pltpu.VMEM((CHUNK,), jnp.int32),
          pltpu.VMEM((CHUNK,), jnp.int32),
          pltpu.VMEM((CHUNK,), jnp.int32),
          pltpu.VMEM((CHUNK,), jnp.int32),
          pltpu.VMEM((CHUNK,), jnp.int32),
          pltpu.VMEM((CHUNK, D), jnp.float32),
          pltpu.VMEM((CHUNK, D), jnp.float32),
          pltpu.VMEM((CHUNK, D), jnp.float32),
          pltpu.SemaphoreType.DMA,
          pltpu.SemaphoreType.DMA,
          pltpu.SemaphoreType.DMA,
          pltpu.SemaphoreType.DMA,
          pltpu.SemaphoreType.DMA,
          pltpu.SemaphoreType.DMA,
      ],
  )
  def k(x_hbm, eyer_hbm, sdm_hbm, acc_out, deg_out,
        t0, t1, t2, td, sd0, sd1, x0, x1, x2, xd, dsh, rows0, rows1, aux,
        semg0, semg1, semi0, semi1, sem1, sems):
    c = lax.axis_index("c")
    s = lax.axis_index("s")
    g = c * NS + s
    tabs = (t0, t1, t2)
    tix = (x0, x1, x2)

    def fill(i, _):
      r = i // (D // 16)
      q = i % (D // 16)
      aux[r, pl.ds(q * 16, 16)] = jnp.zeros((16,), jnp.float32)
      return 0
    lax.fori_loop(0, CHUNK * (D // 16), fill, 0)

    for t in tabs:
      for kk in range(ZC):
        lo = (s * ZC + kk) * CHUNK
        pltpu.async_copy(aux, t.at[pl.ds(lo, CHUNK)], sems)
    pltpu.async_copy(aux.at[pl.ds(0, DZR)], td.at[pl.ds(s * DZR, DZR)], sems)
    for t in tabs:
      for kk in range(ZC):
        lo = (s * ZC + kk) * CHUNK
        pltpu.make_async_copy(aux, t.at[pl.ds(lo, CHUNK)], sems).wait()
    pltpu.make_async_copy(aux.at[pl.ds(0, DZR)],
                          td.at[pl.ds(s * DZR, DZR)], sems).wait()

    plsc.subcore_barrier()

    pltpu.sync_copy(sdm_hbm.at[g, 0], sd0)
    pltpu.async_copy(x_hbm.at[sd0.at[0]], rows0, semg0)
    pltpu.async_copy(sdm_hbm.at[g, 1], sd1, semi1)

    def half(j, sd, rows, semg, sd_n, rows_n, semg_n, semi_n, semi_f):
      for v in range(CHUNK // 16):
        d = sd[1, pl.ds(v * 16, 16)]
        one = jnp.ones((16,), jnp.int32)
        zero = jnp.zeros((16,), jnp.int32)
        tv = (jnp.where(d >= RPT, one, zero)
              + jnp.where(d >= 2 * RPT, one, zero))
        spill = TRASH + (d & 63)
        for t in range(NTB):
          tix[t][pl.ds(v * 16, 16)] = jnp.where(tv == t, d - t * RPT, spill)
        lane = lax.broadcasted_iota(jnp.int32, (16,), 0) + v * 16
        xd[pl.ds(v * 16, 16)] = (d & (D - 1)) + lane * D
        dsh[pl.ds(v * 16, 16)] = d >> (D.bit_length() - 1)
      pltpu.async_copy(eyer_hbm.at[xd], aux, sem1)
      pltpu.make_async_copy(x_hbm.at[sd.at[0]], rows, semg).wait()
      for t in range(NTB):
        pltpu.async_copy(rows, tabs[t].at[tix[t]], sems, add=True)
      pltpu.make_async_copy(sdm_hbm.at[g, j + 1], sd_n, semi_n).wait()
      pltpu.async_copy(x_hbm.at[sd_n.at[0]], rows_n, semg_n)
      pltpu.async_copy(sdm_hbm.at[g, j + 2], sd, semi_f)
      pltpu.make_async_copy(eyer_hbm.at[xd], aux, sem1).wait()
      pltpu.async_copy(aux, td.at[dsh], sems, add=True)
      for t in range(NTB):
        pltpu.make_async_copy(rows, tabs[t].at[tix[t]], sems).wait()
      pltpu.make_async_copy(aux, td.at[dsh], sems).wait()

    def chunk(i, _):
      j0 = 2 * i
      half(j0, sd0, rows0, semg0, sd1, rows1, semg1, semi1, semi0)
      half(j0 + 1, sd1, rows1, semg1, sd0, rows0, semg0, semi0, semi1)
      return 0

    lax.fori_loop(0, NCH // 2, chunk, 0)

    pltpu.make_async_copy(x_hbm.at[sd0.at[0]], rows0, semg0).wait()
    pltpu.make_async_copy(sdm_hbm.at[g, 0], sd1, semi1).wait()

    plsc.subcore_barrier()
    pairs = []
    for t in range(NTB):
      for kk in range(ZC):
        lo = (s * ZC + kk) * CHUNK
        pairs.append((tabs[t].at[pl.ds(lo, CHUNK)],
                      acc_out.at[c, t, pl.ds(lo, CHUNK)]))
    stage = (rows0, rows1)
    sem = (semg0, semg1)
    for i, (src, dst) in enumerate(pairs):
      if i >= 2:
        pltpu.make_async_copy(stage[i % 2], pairs[i - 2][1], sem[i % 2]).wait()
      pltpu.sync_copy(src, stage[i % 2])
      pltpu.async_copy(stage[i % 2], dst, sem[i % 2])
    n = len(pairs)
    pltpu.make_async_copy(stage[n % 2], pairs[n - 2][1], sem[n % 2]).wait()
    pltpu.make_async_copy(stage[(n + 1) % 2], pairs[n - 1][1],
                          sem[(n + 1) % 2]).wait()
    pltpu.sync_copy(td.at[pl.ds(s * DZR, DZR)], aux.at[pl.ds(0, DZR)])
    pltpu.sync_copy(aux.at[pl.ds(0, DZR)], deg_out.at[c, pl.ds(s * DZR, DZR)])

  return k(x, eyer, sdm)


BLK = 1000


def _tc_body(alpha_ref, bs_ref, acc_ref, deg_ref, x_ref, rr_ref,
             wlt_ref, bl_ref, wrt_ref, ws_ref, o_ref):
  aggsum = acc_ref[0, 0] + acc_ref[1, 0]
  degc = jnp.maximum(deg_ref[...], 1.0)
  hl = jnp.dot(aggsum, wlt_ref[...], preferred_element_type=jnp.float32)
  h = (hl / degc + bl_ref[...]
       + jnp.dot(x_ref[...], wrt_ref[...], preferred_element_type=jnp.float32))
  h = jnp.maximum(h, 0.0)
  gnn = jnp.dot(h, ws_ref[...], preferred_element_type=jnp.float32) + bs_ref[0, 0]
  a = 1.0 / (1.0 + jnp.exp(-alpha_ref[0, 0]))
  o_ref[...] = a * rr_ref[...] + (1.0 - a) * gnn


def _tc_dense(alpha, b_s, acc, deg2, x, rr2, wl_t, bl2, wr_t, ws_col):
  grid = (N_NODES // BLK,)
  return pl.pallas_call(
      _tc_body,
      grid=grid,
      in_specs=[
          pl.BlockSpec(memory_space=pltpu.SMEM),
          pl.BlockSpec(memory_space=pltpu.SMEM),
          pl.BlockSpec((NC, 1, BLK, D), lambda i: (0, i // 4, i % 4, 0)),
          pl.BlockSpec((BLK, 1), lambda i: (i, 0)),
          pl.BlockSpec((BLK, D), lambda i: (i, 0)),
          pl.BlockSpec((BLK, 1), lambda i: (i, 0)),
          pl.BlockSpec((D, D), lambda i: (0, 0)),
          pl.BlockSpec((1, D), lambda i: (0, 0)),
          pl.BlockSpec((D, D), lambda i: (0, 0)),
          pl.BlockSpec((D, 1), lambda i: (0, 0)),
      ],
      out_specs=pl.BlockSpec((BLK, 1), lambda i: (i, 0)),
      out_shape=jax.ShapeDtypeStruct((N_NODES, 1), jnp.float32),
  )(alpha, b_s, acc, deg2, x, rr2, wl_t, bl2, wr_t, ws_col)


@jax.jit
def kernel(x, edge_index, reranker_scores, W_l, b_l, W_r, W_s, b_s, alpha):
  src = edge_index[0].astype(jnp.int32)
  dst = edge_index[1].astype(jnp.int32)
  pad = E_PAD - N_EDGES
  pad_ids = jnp.arange(pad, dtype=jnp.int32)
  src_p = jnp.concatenate([src, (pad_ids * 997) % N_NODES])
  dst_p = jnp.concatenate([dst, N_NODES + (pad_ids % 240)])
  sdm = jnp.stack([src_p.reshape(NW, NCH, CHUNK),
                   dst_p.reshape(NW, NCH, CHUNK)], axis=2)
  sdm = jnp.concatenate([sdm, sdm[:, :2]], axis=1)
  eyer = jnp.tile(jnp.eye(D, dtype=jnp.float32), (EREP, 1))

  acc, degp = _sc_segment_sum(x, eyer, sdm)

  deg2 = (degp[0] + degp[1]).reshape(DROWS * D)[:N_NODES].reshape(N_NODES, 1)

  alpha2 = alpha.reshape(1, 1)
  bs2 = b_s.reshape(1, 1)
  rr2 = reranker_scores.reshape(N_NODES, 1)
  out2 = _tc_dense(alpha2, bs2, acc, deg2, x, rr2,
                   W_l.T, b_l.reshape(1, D), W_r.T, W_s.T)
  return out2.reshape(N_NODES)
